# Initial kernel scaffold; baseline (speedup 1.0000x reference)
import jax
import jax.numpy as jnp
from jax.experimental import pallas as pl


def kernel(protein_raw, v_prior, trust_vector, W1, b1, prelu_a, ln_g, ln_b, W2, b2):
    raise NotImplementedError("write your pallas kernel here")


# speedup vs baseline: 2.5977x; 2.5977x over previous
import jax
import jax.numpy as jnp
from jax.experimental import pallas as pl

N = 8192
P, D, T = 1024, 512, 512
H = 512
E = 16
TN = 512


def _gate_kernel(xp_ref, xd_ref, xt_ref, w1_ref, b1_ref, a_ref, g_ref,
                 bb_ref, w2_ref, b2_ref, out_ref):
    dn = (((1,), (1,)), ((), ()))
    h = jax.lax.dot_general(xp_ref[...], w1_ref[:, :P], dn,
                            preferred_element_type=jnp.float32)
    h = h + jax.lax.dot_general(xd_ref[...], w1_ref[:, P:P + D], dn,
                                preferred_element_type=jnp.float32)
    h = h + jax.lax.dot_general(xt_ref[...], w1_ref[:, P + D:], dn,
                                preferred_element_type=jnp.float32)
    h = h + b1_ref[...]
    a = a_ref[0, 0]
    h = jnp.maximum(h, 0.0) + a * jnp.minimum(h, 0.0)
    mu = jnp.mean(h, axis=-1, keepdims=True)
    c = h - mu
    var = jnp.mean(c * c, axis=-1, keepdims=True)
    hn = c * jax.lax.rsqrt(var + 1e-5) * g_ref[...] + bb_ref[...]
    logits = jax.lax.dot_general(hn, w2_ref[...], dn,
                                 preferred_element_type=jnp.float32)
    logits = logits + b2_ref[...]
    iota = jax.lax.broadcasted_iota(jnp.int32, logits.shape, 1)
    m1 = jnp.max(logits, axis=-1, keepdims=True)
    idx1 = jnp.min(jnp.where(logits == m1, iota, E), axis=-1, keepdims=True)
    is1 = iota == idx1
    masked = jnp.where(is1, -jnp.inf, logits)
    m2 = jnp.max(masked, axis=-1, keepdims=True)
    idx2 = jnp.min(jnp.where(masked == m2, iota, E), axis=-1, keepdims=True)
    is2 = iota == idx2
    e2 = jnp.exp(m2 - m1)
    z = 1.0 + e2
    out_ref[...] = jnp.where(is1, 1.0 / z, jnp.where(is2, e2 / z, 0.0))


def kernel(protein_raw, v_prior, trust_vector, W1, b1, prelu_a, ln_g, ln_b,
           W2, b2):
    b1r = b1.reshape(1, H)
    ar = jnp.asarray(prelu_a, jnp.float32).reshape(1, 1)
    gr = ln_g.reshape(1, H)
    br = ln_b.reshape(1, H)
    b2r = b2.reshape(1, E)
    grid = (N // TN,)
    full = lambda i: (0, 0)
    row = lambda i: (i, 0)
    return pl.pallas_call(
        _gate_kernel,
        grid=grid,
        in_specs=[
            pl.BlockSpec((TN, P), row),
            pl.BlockSpec((TN, D), row),
            pl.BlockSpec((TN, T), row),
            pl.BlockSpec((H, P + D + T), full),
            pl.BlockSpec((1, H), full),
            pl.BlockSpec((1, 1), full),
            pl.BlockSpec((1, H), full),
            pl.BlockSpec((1, H), full),
            pl.BlockSpec((E, H), full),
            pl.BlockSpec((1, E), full),
        ],
        out_specs=pl.BlockSpec((TN, E), row),
        out_shape=jax.ShapeDtypeStruct((N, E), jnp.float32),
    )(protein_raw, v_prior, trust_vector, W1, b1r, ar, gr, br, W2, b2r)


# speedup vs baseline: 3.2608x; 1.2553x over previous
import jax
import jax.numpy as jnp
from jax.experimental import pallas as pl

N = 8192
P, D, T = 1024, 512, 512
H = 512
E = 16
TN = 1024


def _gate_kernel(xp_ref, xd_ref, xt_ref, w1_ref, b1_ref, a_ref, g_ref,
                 bb_ref, w2_ref, b2t_ref, out_ref):
    dn = (((1,), (1,)), ((), ()))
    h = jax.lax.dot_general(xp_ref[...], w1_ref[:, :P], dn,
                            preferred_element_type=jnp.float32)
    h = h + jax.lax.dot_general(xd_ref[...], w1_ref[:, P:P + D], dn,
                                preferred_element_type=jnp.float32)
    h = h + jax.lax.dot_general(xt_ref[...], w1_ref[:, P + D:], dn,
                                preferred_element_type=jnp.float32)
    h = h + b1_ref[...]
    a = a_ref[0, 0]
    h = jnp.where(h >= 0.0, h, a * h)
    mu = jnp.mean(h, axis=-1, keepdims=True)
    mu2 = jnp.mean(h * h, axis=-1, keepdims=True)
    s = jax.lax.rsqrt(mu2 - mu * mu + 1e-5)
    hn = (h - mu) * s * g_ref[...] + bb_ref[...]
    lt = jax.lax.dot_general(w2_ref[...], hn, dn,
                             preferred_element_type=jnp.float32)
    lt = lt + b2t_ref[...]
    iota = jax.lax.broadcasted_iota(jnp.int32, lt.shape, 0)
    m1 = jnp.max(lt, axis=0, keepdims=True)
    idx1 = jnp.min(jnp.where(lt == m1, iota, E), axis=0, keepdims=True)
    is1 = iota == idx1
    masked = jnp.where(is1, -jnp.inf, lt)
    m2 = jnp.max(masked, axis=0, keepdims=True)
    idx2 = jnp.min(jnp.where(masked == m2, iota, E), axis=0, keepdims=True)
    e2 = jnp.exp(m2 - m1)
    z = 1.0 + e2
    pt = jnp.where(is1, 1.0 / z, jnp.where(iota == idx2, e2 / z, 0.0))
    out_ref[...] = pt.T


def kernel(protein_raw, v_prior, trust_vector, W1, b1, prelu_a, ln_g, ln_b,
           W2, b2):
    b1r = b1.reshape(1, H)
    ar = jnp.asarray(prelu_a, jnp.float32).reshape(1, 1)
    gr = ln_g.reshape(1, H)
    br = ln_b.reshape(1, H)
    b2t = b2.reshape(E, 1)
    grid = (N // TN,)
    full = lambda i: (0, 0)
    row = lambda i: (i, 0)
    return pl.pallas_call(
        _gate_kernel,
        grid=grid,
        in_specs=[
            pl.BlockSpec((TN, P), row),
            pl.BlockSpec((TN, D), row),
            pl.BlockSpec((TN, T), row),
            pl.BlockSpec((H, P + D + T), full),
            pl.BlockSpec((1, H), full),
            pl.BlockSpec((1, 1), full),
            pl.BlockSpec((1, H), full),
            pl.BlockSpec((1, H), full),
            pl.BlockSpec((E, H), full),
            pl.BlockSpec((E, 1), full),
        ],
        out_specs=pl.BlockSpec((TN, E), row),
        out_shape=jax.ShapeDtypeStruct((N, E), jnp.float32),
    )(protein_raw, v_prior, trust_vector, W1, b1r, ar, gr, br, W2, b2t)
